# baseline (speedup 1.0000x reference)
import functools

import jax
import jax.numpy as jnp
from jax import lax
from jax.experimental import pallas as pl
from jax.experimental.pallas import tpu as pltpu
from jax.experimental.pallas import tpu_sc as plsc

B = 16384
D = 128
LANES = 16
NC = 2
NS = 16
NW = NC * NS
BPW = B // NW
CH = 128
NCHUNK = BPW // CH


def _sc_body(user_h, item_h, theta_h, a_h, b_h, out_h,
             uidx, iidx, tb0, ab0, bb0, tb1, ab1, bb1, obuf, sem0, sem1):
    wid = lax.axis_index("s") * NC + lax.axis_index("c")
    pltpu.sync_copy(user_h.at[wid], uidx)
    pltpu.sync_copy(item_h.at[wid], iidx)

    bufs = ((tb0, ab0, bb0, sem0), (tb1, ab1, bb1, sem1))

    def start(c):
        tb, ab, bb, sem = bufs[c % 2]
        return (pltpu.async_copy(theta_h.at[uidx.at[c]], tb, sem),
                pltpu.async_copy(a_h.at[iidx.at[c]], ab, sem),
                pltpu.async_copy(b_h.at[iidx.at[c]], bb, sem))

    lane = lax.iota(jnp.int32, LANES)
    handles = [start(0)]

    for c in range(NCHUNK):
        if c + 1 < NCHUNK:
            handles.append(start(c + 1))
        for h in handles[c]:
            h.wait()
        tb, ab, bb, _ = bufs[c % 2]

        def group(g, carry, tb=tb, ab=ab, bb=bb, c=c):
            res = jnp.zeros((LANES,), jnp.float32)
            for j in range(LANES):
                s = g * LANES + j
                acc = jnp.zeros((LANES,), jnp.float32)
                for r in range(D // LANES):
                    av = ab[s, pl.ds(r * LANES, LANES)]
                    th = tb[s, pl.ds(r * LANES, LANES)]
                    acc = acc + th / (1.0 + jnp.exp(-av))
                res = jnp.where(lane == j, jnp.sum(acc), res)
            bvec = bb[pl.ds(g * LANES, LANES)]
            logit = res - bvec
            obuf[pl.ds(c * CH + g * LANES, LANES)] = 1.0 / (1.0 + jnp.exp(-logit))
            return carry

        lax.fori_loop(0, CH // LANES, group, 0)

    pltpu.sync_copy(obuf, out_h.at[wid])


def kernel(user, item, theta_w, a_w, b_w):
    user3 = user.reshape(NW, NCHUNK, CH)
    item3 = item.reshape(NW, NCHUNK, CH)
    b_w = b_w.reshape(-1)
    mesh = plsc.VectorSubcoreMesh(core_axis_name="c", subcore_axis_name="s")
    run = pl.kernel(
        _sc_body,
        mesh=mesh,
        out_type=jax.ShapeDtypeStruct((NW, BPW), jnp.float32),
        scratch_types=[
            pltpu.VMEM((NCHUNK, CH), jnp.int32),
            pltpu.VMEM((NCHUNK, CH), jnp.int32),
            pltpu.VMEM((CH, D), jnp.float32),
            pltpu.VMEM((CH, D), jnp.float32),
            pltpu.VMEM((CH,), jnp.float32),
            pltpu.VMEM((CH, D), jnp.float32),
            pltpu.VMEM((CH, D), jnp.float32),
            pltpu.VMEM((CH,), jnp.float32),
            pltpu.VMEM((BPW,), jnp.float32),
            pltpu.SemaphoreType.DMA,
            pltpu.SemaphoreType.DMA,
        ],
        compiler_params=pltpu.CompilerParams(needs_layout_passes=False),
    )
    out = run(user3, item3, theta_w, a_w, b_w)
    return out.reshape(B)
